# Initial kernel scaffold; baseline (speedup 1.0000x reference)
import functools

import jax
import jax.numpy as jnp
from jax import lax
from jax.experimental import pallas as pl
from jax.experimental.pallas import tpu as pltpu
from jax.experimental.pallas import tpu_sc as plsc

N = 10000
E = 320000
D = 128
DG = 64

NC = 2
NS = 16
NW = NC * NS

CHUNK = 128
NCH = -(-E // (NW * CHUNK))
E_PAD = NW * NCH * CHUNK

N_ACC = N + 16
ROWS_INIT = N_ACC // NS
ROWS_OUT = N // NS

_sc_mesh = plsc.VectorSubcoreMesh(core_axis_name="c", subcore_axis_name="s")


@functools.partial(
    pl.kernel,
    out_type=jax.ShapeDtypeStruct((2, N, DG), jnp.float32),
    mesh=_sc_mesh,
    scratch_types=[
        pltpu.VMEM((NCH, CHUNK), jnp.int32),
        pltpu.VMEM((NCH, CHUNK), jnp.int32),
        pltpu.VMEM((CHUNK, DG), jnp.float32),
        pltpu.VMEM_SHARED((N_ACC, DG), jnp.float32),
        pltpu.SemaphoreType.DMA,
    ],
)
def _sc_segment_sum(h_hbm, src_hbm, dst_hbm, zero_hbm, out_hbm,
                    src_v, dst_v, rows_v, acc_sh, sem):
    cid = lax.axis_index("c")
    sid = lax.axis_index("s")
    wid = cid * NS + sid

    row0 = sid * ROWS_INIT
    pltpu.sync_copy(zero_hbm.at[pl.ds(row0, ROWS_INIT)],
                    acc_sh.at[pl.ds(row0, ROWS_INIT)])

    pltpu.sync_copy(src_hbm.at[wid], src_v)
    pltpu.sync_copy(dst_hbm.at[wid], dst_v)
    plsc.subcore_barrier()

    def body(j, carry):
        pltpu.async_copy(h_hbm.at[src_v.at[j]], rows_v, sem).wait()
        pltpu.sync_copy(rows_v, acc_sh.at[dst_v.at[j]], add=True)
        return carry

    lax.fori_loop(0, NCH, body, 0)
    plsc.subcore_barrier()

    out0 = sid * ROWS_OUT
    pltpu.sync_copy(acc_sh.at[pl.ds(out0, ROWS_OUT)],
                    out_hbm.at[cid, pl.ds(out0, ROWS_OUT)])


def _tc_mm_kernel(x_ref, w_ref, h_ref):
    h_ref[...] = jnp.maximum(
        jnp.dot(x_ref[...], w_ref[...], preferred_element_type=jnp.float32), 0.0)


def _tc_add_mm_kernel(x0_ref, p_ref, w_ref, y_ref, h_ref):
    y = x0_ref[...] + p_ref[0] + p_ref[1]
    y_ref[...] = y
    h_ref[...] = jnp.maximum(
        jnp.dot(y, w_ref[...], preferred_element_type=jnp.float32), 0.0)


def _tc_final_kernel(y0_ref, x1_ref, p_ref, out_ref):
    out_ref[:, :DG] = y0_ref[...]
    out_ref[:, DG:] = x1_ref[...] + p_ref[0] + p_ref[1]


_tc_mm = pl.pallas_call(
    _tc_mm_kernel,
    out_shape=jax.ShapeDtypeStruct((N, DG), jnp.float32),
)

_tc_add_mm = pl.pallas_call(
    _tc_add_mm_kernel,
    out_shape=(jax.ShapeDtypeStruct((N, DG), jnp.float32),
               jax.ShapeDtypeStruct((N, DG), jnp.float32)),
)

_tc_final = pl.pallas_call(
    _tc_final_kernel,
    out_shape=jax.ShapeDtypeStruct((N, D), jnp.float32),
)


@jax.jit
def kernel(x, edge_index, W0, W1):
    x0 = x[:, :DG]
    x1 = x[:, DG:]

    pad = E_PAD - E
    src = jnp.concatenate([edge_index[0], jnp.zeros((pad,), jnp.int32)])
    dst = jnp.concatenate([edge_index[1], jnp.full((pad,), N, jnp.int32)])
    src_r = src.reshape(NW, NCH, CHUNK)
    dst_r = dst.reshape(NW, NCH, CHUNK)
    zeros = jnp.zeros((N_ACC, DG), jnp.float32)

    h0 = _tc_mm(x1, W0)
    p0 = _sc_segment_sum(h0, src_r, dst_r, zeros)
    y0, h1 = _tc_add_mm(x0, p0, W1)
    p1 = _sc_segment_sum(h1, src_r, dst_r, zeros)
    return _tc_final(y0, x1, p1)


# speedup vs baseline: 6.9760x; 6.9760x over previous
import functools

import jax
import jax.numpy as jnp
from jax import lax
from jax.experimental import pallas as pl
from jax.experimental.pallas import tpu as pltpu
from jax.experimental.pallas import tpu_sc as plsc

N = 10000
E = 320000
D = 128
DG = 64

NC = 2
NS = 16
NW = NC * NS

CHUNK = 128
NCH = -(-E // (NW * CHUNK))
E_PAD = NW * NCH * CHUNK

N_ACC = 10112
ROWS_ACC = N_ACC // NS

_sc_mesh = plsc.VectorSubcoreMesh(core_axis_name="c", subcore_axis_name="s")


@functools.partial(
    pl.kernel,
    out_type=jax.ShapeDtypeStruct((2, N_ACC, DG), jnp.float32),
    mesh=_sc_mesh,
    scratch_types=[
        pltpu.VMEM((NCH, CHUNK), jnp.int32),
        pltpu.VMEM((NCH, CHUNK), jnp.int32),
        pltpu.VMEM((CHUNK, DG), jnp.float32),
        pltpu.VMEM_SHARED((N_ACC, DG), jnp.float32),
        pltpu.SemaphoreType.DMA,
    ],
    compiler_params=pltpu.CompilerParams(use_tc_tiling_on_sc=False),
)
def _sc_segment_sum(h_hbm, src_hbm, dst_hbm, zero_hbm, out_hbm,
                    src_v, dst_v, rows_v, acc_sh, sem):
    cid = lax.axis_index("c")
    sid = lax.axis_index("s")
    wid = cid * NS + sid

    row0 = sid * ROWS_ACC
    pltpu.sync_copy(zero_hbm.at[pl.ds(row0, ROWS_ACC)],
                    acc_sh.at[pl.ds(row0, ROWS_ACC)])

    pltpu.sync_copy(src_hbm.at[wid], src_v)
    pltpu.sync_copy(dst_hbm.at[wid], dst_v)
    plsc.subcore_barrier()

    def body(j, carry):
        pltpu.async_copy(h_hbm.at[src_v.at[j]], rows_v, sem).wait()
        pltpu.sync_copy(rows_v, acc_sh.at[dst_v.at[j]], add=True)
        return carry

    lax.fori_loop(0, NCH, body, 0)
    plsc.subcore_barrier()

    pltpu.sync_copy(acc_sh.at[pl.ds(row0, ROWS_ACC)],
                    out_hbm.at[cid, pl.ds(row0, ROWS_ACC)])


def _tc_mm_kernel(x_ref, w_ref, h_ref):
    h_ref[...] = jnp.maximum(
        jnp.dot(x_ref[...], w_ref[...], preferred_element_type=jnp.float32), 0.0)


def _tc_add_mm_kernel(x0_ref, p_ref, w_ref, y_ref, h_ref):
    y = x0_ref[...] + p_ref[0, :N] + p_ref[1, :N]
    y_ref[...] = y
    h_ref[...] = jnp.maximum(
        jnp.dot(y, w_ref[...], preferred_element_type=jnp.float32), 0.0)


def _tc_final_kernel(y0_ref, x1_ref, p_ref, out_ref):
    out_ref[:, :DG] = y0_ref[...]
    out_ref[:, DG:] = x1_ref[...] + p_ref[0, :N] + p_ref[1, :N]


_tc_mm = pl.pallas_call(
    _tc_mm_kernel,
    out_shape=jax.ShapeDtypeStruct((N, DG), jnp.float32),
)

_tc_add_mm = pl.pallas_call(
    _tc_add_mm_kernel,
    out_shape=(jax.ShapeDtypeStruct((N, DG), jnp.float32),
               jax.ShapeDtypeStruct((N, DG), jnp.float32)),
)

_tc_final = pl.pallas_call(
    _tc_final_kernel,
    out_shape=jax.ShapeDtypeStruct((N, D), jnp.float32),
)


@jax.jit
def kernel(x, edge_index, W0, W1):
    x0 = x[:, :DG]
    x1 = x[:, DG:]

    pad = E_PAD - E
    src = jnp.concatenate([edge_index[0], jnp.zeros((pad,), jnp.int32)])
    dst = jnp.concatenate([edge_index[1], jnp.full((pad,), N, jnp.int32)])
    src_r = src.reshape(NW, NCH, CHUNK)
    dst_r = dst.reshape(NW, NCH, CHUNK)
    zeros = jnp.zeros((N_ACC, DG), jnp.float32)

    h0 = _tc_mm(x1, W0)
    p0 = _sc_segment_sum(h0, src_r, dst_r, zeros)
    y0, h1 = _tc_add_mm(x0, p0, W1)
    p1 = _sc_segment_sum(h1, src_r, dst_r, zeros)
    return _tc_final(y0, x1, p1)
